# baseline (speedup 1.0000x reference)
import jax
import jax.numpy as jnp
import numpy as np
from jax.experimental import pallas as pl

B, S, HID = 1, 2048, 1024
H, DH = 16, 64
NUM_HASHES = 2
CHUNK = 256
NUM_BUCKETS = 16
N = NUM_HASHES * S
NC = N // CHUNK
KW = 2 * CHUNK
MASK_VAL = -1e9
SELF_MASK_VAL = -1e5


def _proj_kernel(hs_ref, wqk_ref, wv_ref, cos_ref, sin_ref, qk_ref, v_ref):
    hs = hs_ref[...].astype(jnp.bfloat16)
    qk = jnp.dot(hs, wqk_ref[0].astype(jnp.bfloat16),
                 preferred_element_type=jnp.float32)
    v = jnp.dot(hs, wv_ref[0].astype(jnp.bfloat16),
                preferred_element_type=jnp.float32)
    lane = jax.lax.broadcasted_iota(jnp.int32, (S, DH), 1)
    rot2 = jnp.where(lane % 2 == 1, jnp.roll(qk, 1, axis=1),
                     -jnp.roll(qk, -1, axis=1))
    qk_ref[0] = qk * cos_ref[...] + rot2 * sin_ref[...]
    v_ref[0] = v


def _call_proj(hs, W_qk, W_v):
    wqk_h = W_qk.reshape(HID, H, DH).transpose(1, 0, 2)
    wv_h = W_v.reshape(HID, H, DH).transpose(1, 0, 2)
    inv_freq = 1.0 / (10000.0 ** (np.arange(0, DH, 2, dtype=np.float32) / DH))
    sinusoid = np.arange(S, dtype=np.float32)[:, None] * inv_freq[None, :]
    sin = jnp.asarray(np.repeat(np.sin(sinusoid), 2, axis=-1))
    cos = jnp.asarray(np.repeat(np.cos(sinusoid), 2, axis=-1))
    return pl.pallas_call(
        _proj_kernel,
        grid=(H,),
        in_specs=[
            pl.BlockSpec((S, HID), lambda h: (0, 0)),
            pl.BlockSpec((1, HID, DH), lambda h: (h, 0, 0)),
            pl.BlockSpec((1, HID, DH), lambda h: (h, 0, 0)),
            pl.BlockSpec((S, DH), lambda h: (0, 0)),
            pl.BlockSpec((S, DH), lambda h: (0, 0)),
        ],
        out_specs=[
            pl.BlockSpec((1, S, DH), lambda h: (h, 0, 0)),
            pl.BlockSpec((1, S, DH), lambda h: (h, 0, 0)),
        ],
        out_shape=[
            jax.ShapeDtypeStruct((H, S, DH), jnp.float32),
            jax.ShapeDtypeStruct((H, S, DH), jnp.float32),
        ],
    )(hs, wqk_h, wv_h, cos, sin)


def _attn_kernel(qv_ref, qvp_ref, qi_ref, ki_ref, out_ref, lg_ref):
    q = qv_ref[0, 0, :, :DH]
    qp = qvp_ref[0, 0, :, :DH]

    def norm(x):
        x = x.astype(jnp.float32)
        var = jnp.mean(x * x, axis=-1, keepdims=True)
        return x * jax.lax.rsqrt(var + 1e-6) * (DH ** -0.5)

    k = jnp.concatenate([norm(qp), norm(q)], axis=0)
    vadj = jnp.concatenate([qvp_ref[0, 0, :, DH:], qv_ref[0, 0, :, DH:]],
                           axis=0)
    dots = jax.lax.dot_general(q, k.astype(jnp.bfloat16),
                               (((1,), (1,)), ((), ())),
                               preferred_element_type=jnp.float32)
    qi = qi_ref[0, 0]
    ki = ki_ref[0, 0]
    dots = jnp.where(qi >= ki, dots, MASK_VAL)
    dots = jnp.where(qi != ki, dots, SELF_MASK_VAL)
    m = jnp.max(dots, axis=-1, keepdims=True)
    ex = jnp.exp(dots - m)
    ssum = jnp.sum(ex, axis=-1, keepdims=True)
    lg_ref[0, 0] = m + jnp.log(ssum)
    out_ref[0, 0] = jnp.dot((ex / ssum).astype(jnp.bfloat16), vadj,
                            preferred_element_type=jnp.float32
                            ).astype(jnp.bfloat16)


def _rotary_xla(x):
    inv_freq = 1.0 / (10000.0 ** (jnp.arange(0, DH, 2, dtype=jnp.float32) / DH))
    sinusoid = jnp.einsum('i,j->ij', jnp.arange(S, dtype=jnp.float32), inv_freq)
    sin = jnp.repeat(jnp.sin(sinusoid), 2, axis=-1)[None, :, None, :]
    cos = jnp.repeat(jnp.cos(sinusoid), 2, axis=-1)[None, :, None, :]
    x1 = x[..., ::2]
    x2 = x[..., 1::2]
    y = jnp.stack((-x2, x1), axis=-1)
    rot = y.reshape(y.shape[:-2] + (-1,))
    return x * cos + rot * sin


def kernel(hidden_states, W_qk, W_v, rotations):
    hs = hidden_states[0]
    qk, v = _call_proj(hs, W_qk, W_v)

    qk_x = (hidden_states @ W_qk).reshape(B, S, H, DH)
    qk_x = _rotary_xla(qk_x).transpose(0, 2, 1, 3)
    rotated = jnp.einsum('bmtd,mdhr->bmhtr', qk_x, rotations)
    rotated = jnp.concatenate([rotated, -rotated], axis=-1)
    buckets = jnp.argmax(rotated, axis=-1)
    offsets = (jnp.arange(NUM_HASHES) * NUM_BUCKETS).reshape(1, 1, -1, 1)
    bk = (buckets + offsets).reshape(H, N).astype(jnp.int32)

    keys = N * bk + jnp.arange(N, dtype=jnp.int32)[None, :]
    sorted_idx = jnp.argsort(keys, axis=-1)
    undo_idx = jnp.argsort(sorted_idx, axis=-1)
    per_hash = (sorted_idx % S).astype(jnp.int32)

    qkv = jnp.concatenate([qk, v], axis=-1).astype(jnp.bfloat16)
    qv = jnp.take_along_axis(qkv, per_hash[..., None], axis=1)
    qv = qv.reshape(H, NC, CHUNK, 2 * DH)
    idx_c = per_hash.reshape(H, NC, CHUNK)
    qidx = idx_c[..., None]
    kidx = jnp.concatenate([jnp.roll(idx_c, 1, axis=1), idx_c],
                           axis=-1).reshape(H, NC, 1, KW)

    prev = lambda h, c: (h, (c - 1) % NC, 0, 0)
    cur = lambda h, c: (h, c, 0, 0)
    out_s, lg_s = pl.pallas_call(
        _attn_kernel,
        grid=(H, NC),
        in_specs=[
            pl.BlockSpec((1, 1, CHUNK, 2 * DH), cur),
            pl.BlockSpec((1, 1, CHUNK, 2 * DH), prev),
            pl.BlockSpec((1, 1, CHUNK, 1), cur),
            pl.BlockSpec((1, 1, 1, KW), cur),
        ],
        out_specs=[
            pl.BlockSpec((1, 1, CHUNK, DH), cur),
            pl.BlockSpec((1, 1, CHUNK, 1), cur),
        ],
        out_shape=[
            jax.ShapeDtypeStruct((H, NC, CHUNK, DH), jnp.bfloat16),
            jax.ShapeDtypeStruct((H, NC, CHUNK, 1), jnp.float32),
        ],
    )(qv, qv, qidx, kidx)

    out_s = out_s.reshape(H, N, DH)
    lg_s = lg_s.reshape(H, N)
    o = jnp.take_along_axis(out_s, undo_idx[..., None], axis=1)
    lg = jnp.take_along_axis(lg_s, undo_idx, axis=1)
    o = o.astype(jnp.float32).reshape(H, NUM_HASHES, S, DH)
    lg = lg.reshape(H, NUM_HASHES, S)
    w = jax.nn.softmax(lg, axis=1)[..., None]
    o = jnp.sum(o * w, axis=1)
    return o.transpose(1, 0, 2).reshape(B, S, H * DH)


# speedup vs baseline: 1.6990x; 1.6990x over previous
import jax
import jax.numpy as jnp
import numpy as np
from jax.experimental import pallas as pl

B, S, HID = 1, 2048, 1024
H, DH = 16, 64
NUM_HASHES = 2
CHUNK = 256
NUM_BUCKETS = 16
N = NUM_HASHES * S
NC = N // CHUNK
KW = 2 * CHUNK
MASK_VAL = -1e9
SELF_MASK_VAL = -1e5


def _proj_kernel(hs_ref, wqk_ref, wv_ref, cos_ref, sin_ref, qk_ref, v_ref):
    hs = hs_ref[...].astype(jnp.bfloat16)
    qk = jnp.dot(hs, wqk_ref[0].astype(jnp.bfloat16),
                 preferred_element_type=jnp.float32)
    v = jnp.dot(hs, wv_ref[0].astype(jnp.bfloat16),
                preferred_element_type=jnp.float32)
    lane = jax.lax.broadcasted_iota(jnp.int32, (S, DH), 1)
    rot2 = jnp.where(lane % 2 == 1, jnp.roll(qk, 1, axis=1),
                     -jnp.roll(qk, -1, axis=1))
    qk_ref[0] = qk * cos_ref[...] + rot2 * sin_ref[...]
    v_ref[0] = v


def _call_proj(hs, W_qk, W_v):
    wqk_h = W_qk.reshape(HID, H, DH).transpose(1, 0, 2)
    wv_h = W_v.reshape(HID, H, DH).transpose(1, 0, 2)
    inv_freq = 1.0 / (10000.0 ** (np.arange(0, DH, 2, dtype=np.float32) / DH))
    sinusoid = np.arange(S, dtype=np.float32)[:, None] * inv_freq[None, :]
    sin = jnp.asarray(np.repeat(np.sin(sinusoid), 2, axis=-1))
    cos = jnp.asarray(np.repeat(np.cos(sinusoid), 2, axis=-1))
    return pl.pallas_call(
        _proj_kernel,
        grid=(H,),
        in_specs=[
            pl.BlockSpec((S, HID), lambda h: (0, 0)),
            pl.BlockSpec((1, HID, DH), lambda h: (h, 0, 0)),
            pl.BlockSpec((1, HID, DH), lambda h: (h, 0, 0)),
            pl.BlockSpec((S, DH), lambda h: (0, 0)),
            pl.BlockSpec((S, DH), lambda h: (0, 0)),
        ],
        out_specs=[
            pl.BlockSpec((1, S, DH), lambda h: (h, 0, 0)),
            pl.BlockSpec((1, S, DH), lambda h: (h, 0, 0)),
        ],
        out_shape=[
            jax.ShapeDtypeStruct((H, S, DH), jnp.float32),
            jax.ShapeDtypeStruct((H, S, DH), jnp.float32),
        ],
    )(hs, wqk_h, wv_h, cos, sin)


def _attn_kernel(qv_ref, qvp_ref, qi_ref, ki_ref, out_ref, lg_ref):
    q = qv_ref[0, 0, :, :DH]
    qp = qvp_ref[0, 0, :, :DH]

    def norm(x):
        var = jnp.mean(x * x, axis=-1, keepdims=True)
        return x * jax.lax.rsqrt(var + 1e-6) * (DH ** -0.5)

    k = jnp.concatenate([norm(qp), norm(q)], axis=0)
    vadj = jnp.concatenate([qvp_ref[0, 0, :, DH:], qv_ref[0, 0, :, DH:]],
                           axis=0)
    dots = jax.lax.dot_general(q.astype(jnp.bfloat16), k.astype(jnp.bfloat16),
                               (((1,), (1,)), ((), ())),
                               preferred_element_type=jnp.float32)
    qi = qi_ref[0, 0]
    ki = ki_ref[0, 0]
    dots = jnp.where(qi >= ki, dots, MASK_VAL)
    dots = jnp.where(qi != ki, dots, SELF_MASK_VAL)
    m = jnp.max(dots, axis=-1, keepdims=True)
    ex = jnp.exp(dots - m)
    ssum = jnp.sum(ex, axis=-1, keepdims=True)
    lg_ref[0, 0] = m + jnp.log(ssum)
    out_ref[0, 0] = jnp.dot((ex / ssum).astype(jnp.bfloat16),
                            vadj.astype(jnp.bfloat16),
                            preferred_element_type=jnp.float32)


def _rotary_xla(x):
    inv_freq = 1.0 / (10000.0 ** (jnp.arange(0, DH, 2, dtype=jnp.float32) / DH))
    sinusoid = jnp.einsum('i,j->ij', jnp.arange(S, dtype=jnp.float32), inv_freq)
    sin = jnp.repeat(jnp.sin(sinusoid), 2, axis=-1)[None, :, None, :]
    cos = jnp.repeat(jnp.cos(sinusoid), 2, axis=-1)[None, :, None, :]
    x1 = x[..., ::2]
    x2 = x[..., 1::2]
    y = jnp.stack((-x2, x1), axis=-1)
    rot = y.reshape(y.shape[:-2] + (-1,))
    return x * cos + rot * sin


def kernel(hidden_states, W_qk, W_v, rotations):
    hs = hidden_states[0]
    qk, v = _call_proj(hs, W_qk, W_v)

    qk_x = (hidden_states @ W_qk).reshape(B, S, H, DH)
    qk_x = _rotary_xla(qk_x).transpose(0, 2, 1, 3)
    rotated = jnp.einsum('bmtd,mdhr->bmhtr', qk_x, rotations)
    rotated = jnp.concatenate([rotated, -rotated], axis=-1)
    buckets = jnp.argmax(rotated, axis=-1)
    offsets = (jnp.arange(NUM_HASHES) * NUM_BUCKETS).reshape(1, 1, -1, 1)
    bk = (buckets + offsets).reshape(H, N).astype(jnp.int32)

    keys = N * bk + jnp.arange(N, dtype=jnp.int32)[None, :]
    sorted_idx = jnp.argsort(keys, axis=-1)
    undo_idx = jnp.argsort(sorted_idx, axis=-1)
    per_hash = (sorted_idx % S).astype(jnp.int32)

    qkv = jnp.concatenate([qk, v], axis=-1)
    qv = jnp.take_along_axis(qkv, per_hash[..., None], axis=1)
    qv = qv.reshape(H, NC, CHUNK, 2 * DH)
    idx_c = per_hash.reshape(H, NC, CHUNK)
    qidx = idx_c[..., None]
    kidx = jnp.concatenate([jnp.roll(idx_c, 1, axis=1), idx_c],
                           axis=-1).reshape(H, NC, 1, KW)

    prev = lambda h, c: (h, (c - 1) % NC, 0, 0)
    cur = lambda h, c: (h, c, 0, 0)
    out_s, lg_s = pl.pallas_call(
        _attn_kernel,
        grid=(H, NC),
        in_specs=[
            pl.BlockSpec((1, 1, CHUNK, 2 * DH), cur),
            pl.BlockSpec((1, 1, CHUNK, 2 * DH), prev),
            pl.BlockSpec((1, 1, CHUNK, 1), cur),
            pl.BlockSpec((1, 1, 1, KW), cur),
        ],
        out_specs=[
            pl.BlockSpec((1, 1, CHUNK, DH), cur),
            pl.BlockSpec((1, 1, CHUNK, 1), cur),
        ],
        out_shape=[
            jax.ShapeDtypeStruct((H, NC, CHUNK, DH), jnp.float32),
            jax.ShapeDtypeStruct((H, NC, CHUNK, 1), jnp.float32),
        ],
    )(qv, qv, qidx, kidx)

    out_s = out_s.reshape(H, N, DH)
    lg_s = lg_s.reshape(H, N)
    o = jnp.take_along_axis(out_s, undo_idx[..., None], axis=1)
    lg = jnp.take_along_axis(lg_s, undo_idx, axis=1)
    o = o.reshape(H, NUM_HASHES, S, DH)
    lg = lg.reshape(H, NUM_HASHES, S)
    w = jax.nn.softmax(lg, axis=1)[..., None]
    o = jnp.sum(o * w, axis=1)
    return o.transpose(1, 0, 2).reshape(B, S, H * DH)
